# baseline (device time: 54732 ns/iter reference)
import jax
import jax.numpy as jnp
from jax import lax
from jax.experimental import pallas as pl
from jax.experimental.pallas import tpu as pltpu

N_DEV = 16
B, S, D = 2, 256, 1024
DC, H, DH, DR = 64, 16, 64, 32
CHT = 2 * DH
CHO = DH
SCALE = (DH + DR) ** -0.5

BITS_A = [8, 4, 2, 1]
SCR_OFF = [0, 1024, 1536, 1792]
DG = lambda a, b_, dims: lax.dot_general(
    a, b_, (dims, ((), ())), preferred_element_type=jnp.float32)


def _logical(x, y, zlo, zhi):
    w = 2 * y + (x + y - 2 * x * y)
    return 4 * (zlo + 2 * zhi) + w


def kernel(x, Wdkv, Wuk, Wuv, Wq, Wqr, Wkr, Wo):
    def body(x_ref, wdkv_ref, wuk_ref, wuv_ref, wq_ref, wqr_ref, wkr_ref,
             wo_ref, out_ref, accT, scr, stg, ctT, qT, qrT, krT, oT,
             sendA, sendB, rsA, rsB, agA, agB):
        my = lax.axis_index("i")
        w = lax.rem(my, 4)
        z = my // 4
        cx = jnp.logical_or(w == 1, w == 2).astype(jnp.int32)
        cy = (w >= 2).astype(jnp.int32)
        zlo = lax.rem(z, 2)
        zhi = z // 2

        e = 8 * cx + 4 * zlo + 2 * cy + zhi
        e1 = lax.rem(e, 2)
        e2 = lax.rem(e // 2, 2)
        e4 = lax.rem(e // 4, 2)
        e8 = e // 8
        p_x = _logical(1 - cx, cy, zlo, zhi)
        p_y = _logical(cx, 1 - cy, zlo, zhi)
        p_zlo = _logical(cx, cy, 1 - zlo, zhi)
        p_zhi = _logical(cx, cy, zlo, 1 - zhi)
        a0 = 8 * e8 + 4 * e4
        a4 = (e // 4) * 4

        barrier = pltpu.get_barrier_semaphore()
        for nbr in (p_x, p_y, p_zlo, p_zhi):
            pl.semaphore_signal(barrier, inc=1, device_id=(nbr,),
                                device_id_type=pl.DeviceIdType.MESH)
        pl.semaphore_wait(barrier, 4)

        for b in range(B):
            ctT[:, b * S:(b + 1) * S] = DG(wdkv_ref[...], x_ref[b],
                                           ((0,), (1,)))
        ct = ctT[...]
        for h in range(H):
            accT[h * CHT:h * CHT + DH, :] = DG(
                wuk_ref[:, h * DH:(h + 1) * DH], ct, ((0,), (0,)))
            accT[h * CHT + DH:(h + 1) * CHT, :] = DG(
                wuv_ref[:, h * DH:(h + 1) * DH], ct, ((0,), (0,)))

        rs_plan = {
            "A": [([(((e // (2 * b_)) * (2 * b_)) + (1 - lax.rem(e // b_, 2)) * b_, b_)],
                   [(((e // (2 * b_)) * (2 * b_)) + lax.rem(e // b_, 2) * b_, b_)],
                   p) for b_, p in zip(BITS_A, [p_x, p_zlo, p_y, p_zhi])],
            "B": [
                ([(4 * (1 - e4), 4), (8 + 4 * (1 - e4), 4)],
                 [(4 * e4, 4), (8 + 4 * e4, 4)], p_zlo),
                ([(8 * (1 - e8) + 4 * e4, 4)], [(a0, 4)], p_x),
                ([(a0 + (1 - e1), 1), (a0 + 2 + (1 - e1), 1)],
                 [(a0 + e1, 1), (a0 + 2 + e1, 1)], p_zhi),
                ([(a0 + 2 * (1 - e2) + e1, 1)],
                 [(a0 + 2 * e2 + e1, 1)], p_y),
            ],
        }
        cols = {"A": slice(0, S), "B": slice(S, 2 * S)}
        ssem = {"A": sendA, "B": sendB}
        rsem = {"A": rsA, "B": rsB}

        for k in range(4):
            started = []
            for pn in ("A", "B"):
                send_segs, keep_segs, partner = rs_plan[pn][k]
                so = SCR_OFF[k]
                go = 0
                descs = []
                for st, n in send_segs:
                    stg[go:go + n * CHT, cols[pn]] = accT[
                        pl.ds(st * CHT, n * CHT), cols[pn]
                    ].astype(jnp.bfloat16)
                    rdma = pltpu.make_async_remote_copy(
                        src_ref=stg.at[go:go + n * CHT, cols[pn]],
                        dst_ref=scr.at[so:so + n * CHT, cols[pn]],
                        send_sem=ssem[pn],
                        recv_sem=rsem[pn].at[k],
                        device_id=(partner,),
                        device_id_type=pl.DeviceIdType.MESH,
                    )
                    rdma.start()
                    descs.append(rdma)
                    so += n * CHT
                    go += n * CHT
                started.append((pn, descs, keep_segs))
            if k == 0:
                qT[:, 0:S] = DG(wq_ref[...], x_ref[0], ((0,), (1,)))
            elif k == 1:
                qT[:, S:2 * S] = DG(wq_ref[...], x_ref[1], ((0,), (1,)))
            elif k == 2:
                for b in range(B):
                    cs = slice(b * S, (b + 1) * S)
                    qrT[:, cs] = DG(wqr_ref[...], x_ref[b], ((0,), (1,)))
                    krT[:, cs] = DG(wkr_ref[...], x_ref[b], ((0,), (1,)))
            for pn, descs, keep_segs in started:
                for rdma in descs:
                    rdma.wait()
                so = SCR_OFF[k]
                for st, n in keep_segs:
                    accT[pl.ds(st * CHT, n * CHT), cols[pn]] = (
                        accT[pl.ds(st * CHT, n * CHT), cols[pn]]
                        + scr[so:so + n * CHT, cols[pn]].astype(jnp.float32))
                    so += n * CHT

        for b in range(B):
            cs = slice(b * S, (b + 1) * S)
            k_t = accT[pl.ds(e * CHT, DH), cs]
            v_t = accT[pl.ds(e * CHT + DH, DH), cs]
            q_t = qT[pl.ds(e * DH, DH), cs]
            qr_t = qrT[pl.ds(e * DR, DR), cs]
            s = DG(q_t, k_t, ((0,), (0,)))
            s = s + DG(qr_t, krT[:, cs], ((0,), (0,)))
            s = s * SCALE
            m = jnp.max(s, axis=1, keepdims=True)
            pr = jnp.exp(s - m)
            pr = pr / jnp.sum(pr, axis=1, keepdims=True)
            oT[pl.ds(e * CHO, CHO), cs] = DG(
                v_t, pr, ((1,), (1,))).astype(jnp.bfloat16)

        ag_plan = {
            "A": [([((e // L) * L, L)], p)
                  for L, p in zip([1, 2, 4, 8], [p_zhi, p_y, p_zlo, p_x])],
            "B": [
                ([(e, 1)], p_y),
                ([(a4 + e1, 1), (a4 + 2 + e1, 1)], p_zhi),
                ([(a0, 4)], p_x),
                ([(4 * e4, 4), (8 + 4 * e4, 4)], p_zlo),
            ],
        }
        asem = {"A": agA, "B": agB}
        all_ag = []
        for j in range(4):
            stage_descs = []
            for pn in ("A", "B"):
                segs, partner = ag_plan[pn][j]
                for st, n in segs:
                    rdma = pltpu.make_async_remote_copy(
                        src_ref=oT.at[pl.ds(st * CHO, n * CHO), cols[pn]],
                        dst_ref=oT.at[pl.ds(st * CHO, n * CHO), cols[pn]],
                        send_sem=ssem[pn],
                        recv_sem=asem[pn].at[j],
                        device_id=(partner,),
                        device_id_type=pl.DeviceIdType.MESH,
                    )
                    rdma.start()
                    stage_descs.append(rdma)
            all_ag.extend(stage_descs)
            if j < 3:
                for rdma in stage_descs:
                    rdma.wait()
            else:
                part0 = DG(
                    oT[pl.ds(e8 * 512, 512), 0:S].astype(jnp.float32),
                    wo_ref[pl.ds(e8 * 512, 512), :], ((0,), (0,)))
                part1 = DG(
                    oT[pl.ds(e4 * 256, 256), S:2 * S].astype(jnp.float32),
                    wo_ref[pl.ds(e4 * 256, 256), :], ((0,), (0,)))
                part1 = part1 + DG(
                    oT[pl.ds(512 + e4 * 256, 256), S:2 * S].astype(
                        jnp.float32),
                    wo_ref[pl.ds(512 + e4 * 256, 256), :], ((0,), (0,)))
                for rdma in stage_descs:
                    rdma.wait_recv()
                out_ref[0] = part0 + DG(
                    oT[pl.ds((1 - e8) * 512, 512), 0:S].astype(jnp.float32),
                    wo_ref[pl.ds((1 - e8) * 512, 512), :], ((0,), (0,)))
                part1 = part1 + DG(
                    oT[pl.ds((1 - e4) * 256, 256), S:2 * S].astype(
                        jnp.float32),
                    wo_ref[pl.ds((1 - e4) * 256, 256), :], ((0,), (0,)))
                out_ref[1] = part1 + DG(
                    oT[pl.ds(512 + (1 - e4) * 256, 256), S:2 * S].astype(
                        jnp.float32),
                    wo_ref[pl.ds(512 + (1 - e4) * 256, 256), :],
                    ((0,), (0,)))
                for rdma in stage_descs:
                    rdma.wait_send()

    return pl.pallas_call(
        body,
        out_shape=jax.ShapeDtypeStruct((B, S, D), jnp.float32),
        in_specs=[pl.BlockSpec(memory_space=pltpu.VMEM)] * 8,
        out_specs=pl.BlockSpec(memory_space=pltpu.VMEM),
        scratch_shapes=[
            pltpu.VMEM((H * CHT, 2 * S), jnp.float32),
            pltpu.VMEM((1920, 2 * S), jnp.bfloat16),
            pltpu.VMEM((8 * CHT, 2 * S), jnp.bfloat16),
            pltpu.VMEM((DC, 2 * S), jnp.float32),
            pltpu.VMEM((D, 2 * S), jnp.float32),
            pltpu.VMEM((H * DR, 2 * S), jnp.float32),
            pltpu.VMEM((DR, 2 * S), jnp.float32),
            pltpu.VMEM((H * CHO, 2 * S), jnp.bfloat16),
            pltpu.SemaphoreType.DMA,
            pltpu.SemaphoreType.DMA,
            pltpu.SemaphoreType.DMA((4,)),
            pltpu.SemaphoreType.DMA((4,)),
            pltpu.SemaphoreType.DMA((4,)),
            pltpu.SemaphoreType.DMA((4,)),
        ],
        compiler_params=pltpu.CompilerParams(collective_id=0),
    )(x, Wdkv, Wuk, Wuv, Wq, Wqr, Wkr, Wo)


# device time: 54409 ns/iter; 1.0059x vs baseline; 1.0059x over previous
import jax
import jax.numpy as jnp
from jax import lax
from jax.experimental import pallas as pl
from jax.experimental.pallas import tpu as pltpu

N_DEV = 16
B, S, D = 2, 256, 1024
DC, H, DH, DR = 64, 16, 64, 32
CHT = 2 * DH
CHO = DH
SCALE = (DH + DR) ** -0.5

BITS_A = [8, 4, 2, 1]
SCR_OFF = [0, 1024, 1536, 1792]
DG = lambda a, b_, dims: lax.dot_general(
    a, b_, (dims, ((), ())), preferred_element_type=jnp.float32)


def _logical(x, y, zlo, zhi):
    w = 2 * y + (x + y - 2 * x * y)
    return 4 * (zlo + 2 * zhi) + w


def kernel(x, Wdkv, Wuk, Wuv, Wq, Wqr, Wkr, Wo):
    def body(x_ref, wdkv_ref, wuk_ref, wuv_ref, wq_ref, wqr_ref, wkr_ref,
             wo_ref, out_ref, accT, scr, stg, ctT, qT, qrT, krT, oT,
             sendA, sendB, rsA, rsB, agA, agB):
        my = lax.axis_index("i")
        w = lax.rem(my, 4)
        z = my // 4
        cx = jnp.logical_or(w == 1, w == 2).astype(jnp.int32)
        cy = (w >= 2).astype(jnp.int32)
        zlo = lax.rem(z, 2)
        zhi = z // 2

        e = 8 * cx + 4 * zlo + 2 * cy + zhi
        e1 = lax.rem(e, 2)
        e2 = lax.rem(e // 2, 2)
        e4 = lax.rem(e // 4, 2)
        e8 = e // 8
        p_x = _logical(1 - cx, cy, zlo, zhi)
        p_y = _logical(cx, 1 - cy, zlo, zhi)
        p_zlo = _logical(cx, cy, 1 - zlo, zhi)
        p_zhi = _logical(cx, cy, zlo, 1 - zhi)
        a0 = 8 * e8 + 4 * e4
        a4 = (e // 4) * 4

        barrier = pltpu.get_barrier_semaphore()
        for nbr in (p_x, p_y, p_zlo, p_zhi):
            pl.semaphore_signal(barrier, inc=1, device_id=(nbr,),
                                device_id_type=pl.DeviceIdType.MESH)
        pl.semaphore_wait(barrier, 4)

        for b in range(B):
            ctT[:, b * S:(b + 1) * S] = DG(wdkv_ref[...], x_ref[b],
                                           ((0,), (1,)))
        ct = ctT[...]
        for h in range(H):
            accT[h * CHT:h * CHT + DH, :] = DG(
                wuk_ref[:, h * DH:(h + 1) * DH], ct,
                ((0,), (0,))).astype(jnp.bfloat16)
            accT[h * CHT + DH:(h + 1) * CHT, :] = DG(
                wuv_ref[:, h * DH:(h + 1) * DH], ct,
                ((0,), (0,))).astype(jnp.bfloat16)

        rs_plan = {
            "A": [([(((e // (2 * b_)) * (2 * b_)) + (1 - lax.rem(e // b_, 2)) * b_, b_)],
                   [(((e // (2 * b_)) * (2 * b_)) + lax.rem(e // b_, 2) * b_, b_)],
                   p) for b_, p in zip(BITS_A, [p_x, p_zlo, p_y, p_zhi])],
            "B": [
                ([(4 * (1 - e4), 4), (8 + 4 * (1 - e4), 4)],
                 [(4 * e4, 4), (8 + 4 * e4, 4)], p_zlo),
                ([(8 * (1 - e8) + 4 * e4, 4)], [(a0, 4)], p_x),
                ([(a0 + (1 - e1), 1), (a0 + 2 + (1 - e1), 1)],
                 [(a0 + e1, 1), (a0 + 2 + e1, 1)], p_zhi),
                ([(a0 + 2 * (1 - e2) + e1, 1)],
                 [(a0 + 2 * e2 + e1, 1)], p_y),
            ],
        }
        cols = {"A": slice(0, S), "B": slice(S, 2 * S)}
        ssem = {"A": sendA, "B": sendB}
        rsem = {"A": rsA, "B": rsB}

        for k in range(4):
            started = []
            for pn in ("A", "B"):
                send_segs, keep_segs, partner = rs_plan[pn][k]
                so = SCR_OFF[k]
                descs = []
                for st, n in send_segs:
                    rdma = pltpu.make_async_remote_copy(
                        src_ref=accT.at[pl.ds(st * CHT, n * CHT), cols[pn]],
                        dst_ref=scr.at[so:so + n * CHT, cols[pn]],
                        send_sem=ssem[pn],
                        recv_sem=rsem[pn].at[k],
                        device_id=(partner,),
                        device_id_type=pl.DeviceIdType.MESH,
                    )
                    rdma.start()
                    descs.append(rdma)
                    so += n * CHT
                started.append((pn, descs, keep_segs))
            if k == 0:
                qT[:, 0:S] = DG(wq_ref[...], x_ref[0], ((0,), (1,)))
            elif k == 1:
                qT[:, S:2 * S] = DG(wq_ref[...], x_ref[1], ((0,), (1,)))
            elif k == 2:
                for b in range(B):
                    cs = slice(b * S, (b + 1) * S)
                    qrT[:, cs] = DG(wqr_ref[...], x_ref[b], ((0,), (1,)))
                    krT[:, cs] = DG(wkr_ref[...], x_ref[b], ((0,), (1,)))
            for pn, descs, keep_segs in started:
                for rdma in descs:
                    rdma.wait()
                so = SCR_OFF[k]
                for st, n in keep_segs:
                    accT[pl.ds(st * CHT, n * CHT), cols[pn]] = (
                        accT[pl.ds(st * CHT, n * CHT), cols[pn]]
                        + scr[so:so + n * CHT, cols[pn]])
                    so += n * CHT

        for b in range(B):
            cs = slice(b * S, (b + 1) * S)
            k_t = accT[pl.ds(e * CHT, DH), cs].astype(jnp.float32)
            v_t = accT[pl.ds(e * CHT + DH, DH), cs].astype(jnp.float32)
            q_t = qT[pl.ds(e * DH, DH), cs]
            qr_t = qrT[pl.ds(e * DR, DR), cs]
            s = DG(q_t, k_t, ((0,), (0,)))
            s = s + DG(qr_t, krT[:, cs], ((0,), (0,)))
            s = s * SCALE
            m = jnp.max(s, axis=1, keepdims=True)
            pr = jnp.exp(s - m)
            pr = pr / jnp.sum(pr, axis=1, keepdims=True)
            oT[pl.ds(e * CHO, CHO), cs] = DG(
                v_t, pr, ((1,), (1,))).astype(jnp.bfloat16)

        ag_plan = {
            "A": [([((e // L) * L, L)], p)
                  for L, p in zip([1, 2, 4, 8], [p_zhi, p_y, p_zlo, p_x])],
            "B": [
                ([(e, 1)], p_y),
                ([(a4 + e1, 1), (a4 + 2 + e1, 1)], p_zhi),
                ([(a0, 4)], p_x),
                ([(4 * e4, 4), (8 + 4 * e4, 4)], p_zlo),
            ],
        }
        asem = {"A": agA, "B": agB}
        all_ag = []
        for j in range(4):
            stage_descs = []
            for pn in ("A", "B"):
                segs, partner = ag_plan[pn][j]
                for st, n in segs:
                    rdma = pltpu.make_async_remote_copy(
                        src_ref=oT.at[pl.ds(st * CHO, n * CHO), cols[pn]],
                        dst_ref=oT.at[pl.ds(st * CHO, n * CHO), cols[pn]],
                        send_sem=ssem[pn],
                        recv_sem=asem[pn].at[j],
                        device_id=(partner,),
                        device_id_type=pl.DeviceIdType.MESH,
                    )
                    rdma.start()
                    stage_descs.append(rdma)
            all_ag.extend(stage_descs)
            if j < 3:
                for rdma in stage_descs:
                    rdma.wait()
            else:
                part0 = DG(
                    oT[pl.ds(e8 * 512, 512), 0:S].astype(jnp.float32),
                    wo_ref[pl.ds(e8 * 512, 512), :], ((0,), (0,)))
                part1 = DG(
                    oT[pl.ds(e4 * 256, 256), S:2 * S].astype(jnp.float32),
                    wo_ref[pl.ds(e4 * 256, 256), :], ((0,), (0,)))
                part1 = part1 + DG(
                    oT[pl.ds(512 + e4 * 256, 256), S:2 * S].astype(
                        jnp.float32),
                    wo_ref[pl.ds(512 + e4 * 256, 256), :], ((0,), (0,)))
                for rdma in stage_descs:
                    rdma.wait_recv()
                out_ref[0] = part0 + DG(
                    oT[pl.ds((1 - e8) * 512, 512), 0:S].astype(jnp.float32),
                    wo_ref[pl.ds((1 - e8) * 512, 512), :], ((0,), (0,)))
                part1 = part1 + DG(
                    oT[pl.ds((1 - e4) * 256, 256), S:2 * S].astype(
                        jnp.float32),
                    wo_ref[pl.ds((1 - e4) * 256, 256), :], ((0,), (0,)))
                out_ref[1] = part1 + DG(
                    oT[pl.ds(512 + (1 - e4) * 256, 256), S:2 * S].astype(
                        jnp.float32),
                    wo_ref[pl.ds(512 + (1 - e4) * 256, 256), :],
                    ((0,), (0,)))
                for rdma in stage_descs:
                    rdma.wait_send()

    return pl.pallas_call(
        body,
        out_shape=jax.ShapeDtypeStruct((B, S, D), jnp.float32),
        in_specs=[pl.BlockSpec(memory_space=pltpu.VMEM)] * 8,
        out_specs=pl.BlockSpec(memory_space=pltpu.VMEM),
        scratch_shapes=[
            pltpu.VMEM((H * CHT, 2 * S), jnp.bfloat16),
            pltpu.VMEM((1920, 2 * S), jnp.bfloat16),
            pltpu.VMEM((8 * CHT, 2 * S), jnp.bfloat16),
            pltpu.VMEM((DC, 2 * S), jnp.float32),
            pltpu.VMEM((D, 2 * S), jnp.float32),
            pltpu.VMEM((H * DR, 2 * S), jnp.float32),
            pltpu.VMEM((DR, 2 * S), jnp.float32),
            pltpu.VMEM((H * CHO, 2 * S), jnp.bfloat16),
            pltpu.SemaphoreType.DMA,
            pltpu.SemaphoreType.DMA,
            pltpu.SemaphoreType.DMA((4,)),
            pltpu.SemaphoreType.DMA((4,)),
            pltpu.SemaphoreType.DMA((4,)),
            pltpu.SemaphoreType.DMA((4,)),
        ],
        compiler_params=pltpu.CompilerParams(collective_id=0),
    )(x, Wdkv, Wuk, Wuv, Wq, Wqr, Wkr, Wo)
